# baseline (speedup 1.0000x reference)
import jax
import jax.numpy as jnp
import numpy as np
from jax import lax
from jax.experimental import pallas as pl
from jax.experimental.pallas import tpu as pltpu
from jax.experimental.pallas import tpu_sc as plsc

N = 10000
E = 320000
D = 128
H = 128
C = 64
G = 64

NC = 2
NS = 16
NW = NC * NS
HP = H // 2

CH = 128
NCHK = 80
QUOTA = NCHK * CH
EP = QUOTA * NW
GSZ = 16
NG = NCHK // GSZ
NBUF = 3
PF = 2

WB = 632
AGG_ROWS = 10008

R = 1000
GRID = N // R

_PERM = np.concatenate(
    [np.concatenate([32 * c + 2 * np.arange(16),
                     32 * c + 2 * np.arange(16) + 1]) for c in range(4)])

_mesh = plsc.VectorSubcoreMesh(
    core_axis_name="c", subcore_axis_name="s", num_cores=NC, num_subcores=NS)


def _sc_body(y_hbm, src_hbm, dst_hbm, out_hbm, src_v, dst_v, pk, fbuf, agg,
             sem_g, sem_i, sem_s):
    c = lax.axis_index("c")
    s = lax.axis_index("s")
    w = c * NS + s

    def _zrow(i, carry):
        for j in range(H // 16):
            fbuf[i, pl.ds(j * 16, 16)] = jnp.zeros((16,), jnp.float32)
        return carry
    lax.fori_loop(0, CH, _zrow, 0)
    for i in range(4):
        pltpu.sync_copy(fbuf, agg.at[pl.ds(s * WB + i * CH, CH)])
    pltpu.sync_copy(fbuf.at[pl.ds(0, 120)],
                    agg.at[pl.ds(s * WB + 4 * CH, 120)])

    @pl.when(s == NS - 1)
    def _():
        pltpu.sync_copy(fbuf.at[pl.ds(0, 8)], agg.at[pl.ds(N, 8)])

    pltpu.async_copy(src_hbm.at[pl.ds(w * NCHK, GSZ)], src_v.at[0], sem_i)
    pltpu.async_copy(dst_hbm.at[pl.ds(w * NCHK * 2, GSZ * 2)], dst_v.at[0],
                     sem_i)
    plsc.subcore_barrier()

    for g in range(NG):
        p = g % 2
        pltpu.make_async_copy(
            src_hbm.at[pl.ds(w * NCHK + g * GSZ, GSZ)], src_v.at[p],
            sem_i).wait()
        pltpu.make_async_copy(
            dst_hbm.at[pl.ds(w * NCHK * 2 + g * GSZ * 2, GSZ * 2)],
            dst_v.at[p], sem_i).wait()
        if g + 1 < NG:
            pltpu.async_copy(
                src_hbm.at[pl.ds(w * NCHK + (g + 1) * GSZ, GSZ)],
                src_v.at[1 - p], sem_i)
            pltpu.async_copy(
                dst_hbm.at[pl.ds(w * NCHK * 2 + (g + 1) * GSZ * 2, GSZ * 2)],
                dst_v.at[1 - p], sem_i)
        for f in range(PF):
            pltpu.async_copy(y_hbm.at[src_v.at[p, f]], pk.at[f], sem_g)

        def _chunk(j, carry):
            jp = j % NBUF

            @pl.when(j + PF < GSZ)
            def _():
                pltpu.async_copy(y_hbm.at[src_v.at[p, j + PF]],
                                 pk.at[(j + PF) % NBUF], sem_g)

            pltpu.make_async_copy(
                y_hbm.at[src_v.at[p, j]], pk.at[jp], sem_g).wait()

            def _unpack_half(base):
                def _cv(i, carry2):
                    for rr in range(2):
                        ii = base + i * 2 + rr
                        for cc in range(4):
                            v = pk[jp, ii, pl.ds(cc * 16, 16)]
                            fbuf[ii, pl.ds(32 * cc, 16)] = plsc.bitcast(
                                v << 16, jnp.float32)
                            fbuf[ii, pl.ds(32 * cc + 16, 16)] = plsc.bitcast(
                                v & jnp.int32(-65536), jnp.float32)
                    return carry2
                lax.fori_loop(0, CH // 4, _cv, 0)

            def _drain_half():
                pltpu.make_async_copy(
                    fbuf.at[pl.ds(0, CH // 2)],
                    agg.at[dst_v.at[0, 0]], sem_s).wait()

            if g == 0:
                @pl.when(j > 0)
                def _():
                    _drain_half()
            else:
                _drain_half()
            _unpack_half(0)
            pltpu.async_copy(fbuf.at[pl.ds(0, CH // 2)],
                             agg.at[dst_v.at[p, 2 * j]], sem_s, add=True)
            if g == 0:
                @pl.when(j > 0)
                def _():
                    _drain_half()
            else:
                _drain_half()
            _unpack_half(CH // 2)
            pltpu.async_copy(fbuf.at[pl.ds(CH // 2, CH // 2)],
                             agg.at[dst_v.at[p, 2 * j + 1]], sem_s, add=True)
            return carry
        lax.fori_loop(0, GSZ, _chunk, 0)
    for _ in range(2):
        pltpu.make_async_copy(fbuf.at[pl.ds(0, CH // 2)],
                              agg.at[dst_v.at[0, 0]], sem_s).wait()
    plsc.subcore_barrier()

    base = s * WB
    pltpu.sync_copy(agg.at[pl.ds(base, 520)],
                    out_hbm.at[pl.ds(c * N + base, 520)])

    @pl.when(s < NS - 1)
    def _():
        pltpu.sync_copy(agg.at[pl.ds(base + 520, 112)],
                        out_hbm.at[pl.ds(c * N + base + 520, 112)])


def _sc_agg(y_pk, src2, dst2):
    fn = pl.kernel(
        _sc_body,
        out_type=jax.ShapeDtypeStruct((NC * N, H), jnp.float32),
        mesh=_mesh,
        compiler_params=pltpu.CompilerParams(
            use_tc_tiling_on_sc=False, needs_layout_passes=False),
        scratch_types=[
            pltpu.VMEM((2, GSZ, CH), jnp.int32),
            pltpu.VMEM((2, GSZ * 2, CH // 2), jnp.int32),
            pltpu.VMEM((NBUF, CH, HP), jnp.int32),
            pltpu.VMEM((CH, H), jnp.float32),
            pltpu.VMEM_SHARED((AGG_ROWS, H), jnp.float32),
            pltpu.SemaphoreType.DMA,
            pltpu.SemaphoreType.DMA,
            pltpu.SemaphoreType.DMA,
        ],
    )
    return fn(y_pk, src2, dst2)


def _pack(y16):
    return jax.lax.bitcast_convert_type(y16.reshape(N, HP, 2), jnp.int32)


def _mm2_body(x_ref, wa_ref, wb_ref, y_ref, r_ref):
    xb = x_ref[...]
    y_ref[...] = jnp.dot(
        xb, wa_ref[...], preferred_element_type=jnp.float32
    ).astype(jnp.bfloat16)
    r_ref[...] = jnp.dot(xb, wb_ref[...], preferred_element_type=jnp.float32)


def _mm2(x, wa, wb):
    return pl.pallas_call(
        _mm2_body,
        grid=(GRID,),
        in_specs=[pl.BlockSpec((R, D), lambda i: (i, 0)),
                  pl.BlockSpec((D, H), lambda i: (0, 0)),
                  pl.BlockSpec((D, H), lambda i: (0, 0))],
        out_specs=[pl.BlockSpec((R, H), lambda i: (i, 0)),
                   pl.BlockSpec((R, H), lambda i: (i, 0))],
        out_shape=[jax.ShapeDtypeStruct((N, H), jnp.bfloat16),
                   jax.ShapeDtypeStruct((N, H), jnp.float32)],
    )(x, wa, wb)


def _combine_body(pa_ref, pb_ref, r_ref, b_ref, wa_ref, wb_ref, y_ref,
                  rn_ref):
    h = jnp.maximum(pa_ref[...] + pb_ref[...] + r_ref[...] + b_ref[...], 0.0)
    y_ref[...] = jnp.dot(
        h, wa_ref[...], preferred_element_type=jnp.float32
    ).astype(jnp.bfloat16)
    rn_ref[...] = jnp.dot(h, wb_ref[...], preferred_element_type=jnp.float32)


def _combine(p, r, b, wa, wb):
    return pl.pallas_call(
        _combine_body,
        grid=(GRID,),
        in_specs=[pl.BlockSpec((R, H), lambda i: (i, 0)),
                  pl.BlockSpec((R, H), lambda i: (i + GRID, 0)),
                  pl.BlockSpec((R, H), lambda i: (i, 0)),
                  pl.BlockSpec((1, H), lambda i: (0, 0)),
                  pl.BlockSpec((H, H), lambda i: (0, 0)),
                  pl.BlockSpec((H, H), lambda i: (0, 0))],
        out_specs=[pl.BlockSpec((R, H), lambda i: (i, 0)),
                   pl.BlockSpec((R, H), lambda i: (i, 0))],
        out_shape=[jax.ShapeDtypeStruct((N, H), jnp.bfloat16),
                   jax.ShapeDtypeStruct((N, H), jnp.float32)],
    )(p, p, r, b, wa, wb)


def _final_body(pa_ref, pb_ref, r_ref, b_ref, batch_ref, wl_ref, bl_ref,
                out_ref, pooled):
    i = pl.program_id(0)
    h = jnp.maximum(pa_ref[...] + pb_ref[...] + r_ref[...] + b_ref[...], 0.0)
    bb = batch_ref[0, 0, :]
    oh = (lax.broadcasted_iota(jnp.int32, (G, R), 0) == bb[None, :]
          ).astype(jnp.float32)
    contrib = jnp.dot(oh, h, preferred_element_type=jnp.float32)

    @pl.when(i == 0)
    def _():
        pooled[...] = contrib

    @pl.when(i > 0)
    def _():
        pooled[...] += contrib

    @pl.when(i == GRID - 1)
    def _():
        logits = jnp.dot(pooled[...], wl_ref[...],
                         preferred_element_type=jnp.float32) + bl_ref[...]
        m = jnp.max(logits, axis=-1, keepdims=True)
        lse = jnp.log(jnp.sum(jnp.exp(logits - m), axis=-1, keepdims=True)) + m
        out_ref[...] = logits - lse


def _final(p, r, b, batch3, wl, bl):
    return pl.pallas_call(
        _final_body,
        grid=(GRID,),
        in_specs=[pl.BlockSpec((R, H), lambda i: (i, 0)),
                  pl.BlockSpec((R, H), lambda i: (i + GRID, 0)),
                  pl.BlockSpec((R, H), lambda i: (i, 0)),
                  pl.BlockSpec((1, H), lambda i: (0, 0)),
                  pl.BlockSpec((1, 1, R), lambda i: (i, 0, 0)),
                  pl.BlockSpec((H, C), lambda i: (0, 0)),
                  pl.BlockSpec((1, C), lambda i: (0, 0))],
        out_specs=pl.BlockSpec((G, C), lambda i: (0, 0)),
        out_shape=jax.ShapeDtypeStruct((G, C), jnp.float32),
        scratch_shapes=[pltpu.VMEM((G, H), jnp.float32)],
    )(p, p, r, b, batch3, wl, bl)


def kernel(x, edge_index, batch,
           W_rel0, b_rel0, W_root0,
           W_rel1, b_rel1, W_root1,
           W_rel2, b_rel2, W_root2,
           W_lin2, b_lin2):
    f32 = jnp.float32
    x = x.astype(f32)
    src = edge_index[0].astype(jnp.int32)
    dst = edge_index[1].astype(jnp.int32)
    pad = EP - E
    src2 = jnp.concatenate([src, jnp.zeros((pad,), jnp.int32)]).reshape(
        EP // CH, CH)
    dst2 = jnp.concatenate([dst, jnp.full((pad,), N, jnp.int32)]).reshape(
        EP // (CH // 2), CH // 2)
    batch3 = batch.astype(jnp.int32).reshape(GRID, 1, R)

    perm = jnp.asarray(_PERM)
    wrel0 = W_rel0.astype(f32)
    wroot0 = W_root0.astype(f32)[:, perm]
    b0 = b_rel0.astype(f32)[perm].reshape(1, H)
    wrel1 = W_rel1.astype(f32)[perm, :]
    wroot1 = W_root1.astype(f32)[perm][:, perm]
    b1 = b_rel1.astype(f32)[perm].reshape(1, H)
    wrel2 = W_rel2.astype(f32)[perm, :]
    wroot2 = W_root2.astype(f32)[perm][:, perm]
    b2 = b_rel2.astype(f32)[perm].reshape(1, H)
    wlin2 = W_lin2.astype(f32)[perm, :]
    bl = b_lin2.astype(f32).reshape(1, C)

    y16, r = _mm2(x, wrel0, wroot0)
    p = _sc_agg(_pack(y16), src2, dst2)
    y16, r = _combine(p, r, b0, wrel1, wroot1)
    p = _sc_agg(_pack(y16), src2, dst2)
    y16, r = _combine(p, r, b1, wrel2, wroot2)
    p = _sc_agg(_pack(y16), src2, dst2)
    return _final(p, r, b2, batch3, wlin2, bl)


# speedup vs baseline: 1.0013x; 1.0013x over previous
import jax
import jax.numpy as jnp
import numpy as np
from jax import lax
from jax.experimental import pallas as pl
from jax.experimental.pallas import tpu as pltpu
from jax.experimental.pallas import tpu_sc as plsc

N = 10000
E = 320000
D = 128
H = 128
C = 64
G = 64

NC = 2
NS = 16
NW = NC * NS
HP = H // 2

CH = 128
NCHK = 80
QUOTA = NCHK * CH
EP = QUOTA * NW
GSZ = 16
NG = NCHK // GSZ
NBUF = 3
PF = 2

WB = 632
AGG_ROWS = 10008

R = 1000
GRID = N // R

_PERM = np.concatenate(
    [np.concatenate([32 * c + 2 * np.arange(16),
                     32 * c + 2 * np.arange(16) + 1]) for c in range(4)])

_mesh = plsc.VectorSubcoreMesh(
    core_axis_name="c", subcore_axis_name="s", num_cores=NC, num_subcores=NS)


def _sc_body(y_hbm, src_hbm, dst_hbm, out_hbm, src_v, dst_v, pk, fbuf, agg,
             sem_g, sem_i, sem_s):
    c = lax.axis_index("c")
    s = lax.axis_index("s")
    w = c * NS + s

    def _zrow(i, carry):
        for j in range(H // 16):
            fbuf[i, pl.ds(j * 16, 16)] = jnp.zeros((16,), jnp.float32)
        return carry
    lax.fori_loop(0, CH, _zrow, 0)
    for i in range(4):
        pltpu.async_copy(fbuf, agg.at[pl.ds(s * WB + i * CH, CH)], sem_s)
    pltpu.async_copy(fbuf.at[pl.ds(0, 120)],
                     agg.at[pl.ds(s * WB + 4 * CH, 120)], sem_s)

    @pl.when(s == NS - 1)
    def _():
        pltpu.sync_copy(fbuf.at[pl.ds(0, 8)], agg.at[pl.ds(N, 8)])
    for i in range(4):
        pltpu.make_async_copy(
            fbuf, agg.at[pl.ds(s * WB + i * CH, CH)], sem_s).wait()
    pltpu.make_async_copy(fbuf.at[pl.ds(0, 120)],
                          agg.at[pl.ds(s * WB + 4 * CH, 120)], sem_s).wait()

    pltpu.async_copy(src_hbm.at[pl.ds(w * NCHK, GSZ)], src_v.at[0], sem_i)
    pltpu.async_copy(dst_hbm.at[pl.ds(w * NCHK * 2, GSZ * 2)], dst_v.at[0],
                     sem_i)
    plsc.subcore_barrier()

    for g in range(NG):
        p = g % 2
        pltpu.make_async_copy(
            src_hbm.at[pl.ds(w * NCHK + g * GSZ, GSZ)], src_v.at[p],
            sem_i).wait()
        pltpu.make_async_copy(
            dst_hbm.at[pl.ds(w * NCHK * 2 + g * GSZ * 2, GSZ * 2)],
            dst_v.at[p], sem_i).wait()
        if g + 1 < NG:
            pltpu.async_copy(
                src_hbm.at[pl.ds(w * NCHK + (g + 1) * GSZ, GSZ)],
                src_v.at[1 - p], sem_i)
            pltpu.async_copy(
                dst_hbm.at[pl.ds(w * NCHK * 2 + (g + 1) * GSZ * 2, GSZ * 2)],
                dst_v.at[1 - p], sem_i)
        for f in range(PF):
            pltpu.async_copy(y_hbm.at[src_v.at[p, f]], pk.at[f], sem_g)

        def _chunk(j, carry):
            jp = j % NBUF

            @pl.when(j + PF < GSZ)
            def _():
                pltpu.async_copy(y_hbm.at[src_v.at[p, j + PF]],
                                 pk.at[(j + PF) % NBUF], sem_g)

            pltpu.make_async_copy(
                y_hbm.at[src_v.at[p, j]], pk.at[jp], sem_g).wait()

            def _unpack_half(base):
                def _cv(i, carry2):
                    for rr in range(2):
                        ii = base + i * 2 + rr
                        for cc in range(4):
                            v = pk[jp, ii, pl.ds(cc * 16, 16)]
                            fbuf[ii, pl.ds(32 * cc, 16)] = plsc.bitcast(
                                v << 16, jnp.float32)
                            fbuf[ii, pl.ds(32 * cc + 16, 16)] = plsc.bitcast(
                                v & jnp.int32(-65536), jnp.float32)
                    return carry2
                lax.fori_loop(0, CH // 4, _cv, 0)

            def _drain_half():
                pltpu.make_async_copy(
                    fbuf.at[pl.ds(0, CH // 2)],
                    agg.at[dst_v.at[0, 0]], sem_s).wait()

            if g == 0:
                @pl.when(j > 0)
                def _():
                    _drain_half()
            else:
                _drain_half()
            _unpack_half(0)
            pltpu.async_copy(fbuf.at[pl.ds(0, CH // 2)],
                             agg.at[dst_v.at[p, 2 * j]], sem_s, add=True)
            if g == 0:
                @pl.when(j > 0)
                def _():
                    _drain_half()
            else:
                _drain_half()
            _unpack_half(CH // 2)
            pltpu.async_copy(fbuf.at[pl.ds(CH // 2, CH // 2)],
                             agg.at[dst_v.at[p, 2 * j + 1]], sem_s, add=True)
            return carry
        lax.fori_loop(0, GSZ, _chunk, 0)
    for _ in range(2):
        pltpu.make_async_copy(fbuf.at[pl.ds(0, CH // 2)],
                              agg.at[dst_v.at[0, 0]], sem_s).wait()
    plsc.subcore_barrier()

    base = s * WB
    pltpu.async_copy(agg.at[pl.ds(base, 520)],
                     out_hbm.at[pl.ds(c * N + base, 520)], sem_g)

    @pl.when(s < NS - 1)
    def _():
        pltpu.sync_copy(agg.at[pl.ds(base + 520, 112)],
                        out_hbm.at[pl.ds(c * N + base + 520, 112)])
    pltpu.make_async_copy(agg.at[pl.ds(base, 520)],
                          out_hbm.at[pl.ds(c * N + base, 520)], sem_g).wait()


def _sc_agg(y_pk, src2, dst2):
    fn = pl.kernel(
        _sc_body,
        out_type=jax.ShapeDtypeStruct((NC * N, H), jnp.float32),
        mesh=_mesh,
        compiler_params=pltpu.CompilerParams(
            use_tc_tiling_on_sc=False, needs_layout_passes=False),
        scratch_types=[
            pltpu.VMEM((2, GSZ, CH), jnp.int32),
            pltpu.VMEM((2, GSZ * 2, CH // 2), jnp.int32),
            pltpu.VMEM((NBUF, CH, HP), jnp.int32),
            pltpu.VMEM((CH, H), jnp.float32),
            pltpu.VMEM_SHARED((AGG_ROWS, H), jnp.float32),
            pltpu.SemaphoreType.DMA,
            pltpu.SemaphoreType.DMA,
            pltpu.SemaphoreType.DMA,
        ],
    )
    return fn(y_pk, src2, dst2)


def _pack(y16):
    return jax.lax.bitcast_convert_type(y16.reshape(N, HP, 2), jnp.int32)


def _mm2_body(x_ref, wa_ref, wb_ref, y_ref, r_ref):
    xb = x_ref[...]
    y_ref[...] = jnp.dot(
        xb, wa_ref[...], preferred_element_type=jnp.float32
    ).astype(jnp.bfloat16)
    r_ref[...] = jnp.dot(xb, wb_ref[...], preferred_element_type=jnp.float32)


def _mm2(x, wa, wb):
    return pl.pallas_call(
        _mm2_body,
        grid=(GRID,),
        in_specs=[pl.BlockSpec((R, D), lambda i: (i, 0)),
                  pl.BlockSpec((D, H), lambda i: (0, 0)),
                  pl.BlockSpec((D, H), lambda i: (0, 0))],
        out_specs=[pl.BlockSpec((R, H), lambda i: (i, 0)),
                   pl.BlockSpec((R, H), lambda i: (i, 0))],
        out_shape=[jax.ShapeDtypeStruct((N, H), jnp.bfloat16),
                   jax.ShapeDtypeStruct((N, H), jnp.float32)],
    )(x, wa, wb)


def _combine_body(pa_ref, pb_ref, r_ref, b_ref, wa_ref, wb_ref, y_ref,
                  rn_ref):
    h = jnp.maximum(pa_ref[...] + pb_ref[...] + r_ref[...] + b_ref[...], 0.0)
    y_ref[...] = jnp.dot(
        h, wa_ref[...], preferred_element_type=jnp.float32
    ).astype(jnp.bfloat16)
    rn_ref[...] = jnp.dot(h, wb_ref[...], preferred_element_type=jnp.float32)


def _combine(p, r, b, wa, wb):
    return pl.pallas_call(
        _combine_body,
        grid=(GRID,),
        in_specs=[pl.BlockSpec((R, H), lambda i: (i, 0)),
                  pl.BlockSpec((R, H), lambda i: (i + GRID, 0)),
                  pl.BlockSpec((R, H), lambda i: (i, 0)),
                  pl.BlockSpec((1, H), lambda i: (0, 0)),
                  pl.BlockSpec((H, H), lambda i: (0, 0)),
                  pl.BlockSpec((H, H), lambda i: (0, 0))],
        out_specs=[pl.BlockSpec((R, H), lambda i: (i, 0)),
                   pl.BlockSpec((R, H), lambda i: (i, 0))],
        out_shape=[jax.ShapeDtypeStruct((N, H), jnp.bfloat16),
                   jax.ShapeDtypeStruct((N, H), jnp.float32)],
    )(p, p, r, b, wa, wb)


def _final_body(pa_ref, pb_ref, r_ref, b_ref, batch_ref, wl_ref, bl_ref,
                out_ref, pooled):
    i = pl.program_id(0)
    h = jnp.maximum(pa_ref[...] + pb_ref[...] + r_ref[...] + b_ref[...], 0.0)
    bb = batch_ref[0, 0, :]
    oh = (lax.broadcasted_iota(jnp.int32, (G, R), 0) == bb[None, :]
          ).astype(jnp.float32)
    contrib = jnp.dot(oh, h, preferred_element_type=jnp.float32)

    @pl.when(i == 0)
    def _():
        pooled[...] = contrib

    @pl.when(i > 0)
    def _():
        pooled[...] += contrib

    @pl.when(i == GRID - 1)
    def _():
        logits = jnp.dot(pooled[...], wl_ref[...],
                         preferred_element_type=jnp.float32) + bl_ref[...]
        m = jnp.max(logits, axis=-1, keepdims=True)
        lse = jnp.log(jnp.sum(jnp.exp(logits - m), axis=-1, keepdims=True)) + m
        out_ref[...] = logits - lse


def _final(p, r, b, batch3, wl, bl):
    return pl.pallas_call(
        _final_body,
        grid=(GRID,),
        in_specs=[pl.BlockSpec((R, H), lambda i: (i, 0)),
                  pl.BlockSpec((R, H), lambda i: (i + GRID, 0)),
                  pl.BlockSpec((R, H), lambda i: (i, 0)),
                  pl.BlockSpec((1, H), lambda i: (0, 0)),
                  pl.BlockSpec((1, 1, R), lambda i: (i, 0, 0)),
                  pl.BlockSpec((H, C), lambda i: (0, 0)),
                  pl.BlockSpec((1, C), lambda i: (0, 0))],
        out_specs=pl.BlockSpec((G, C), lambda i: (0, 0)),
        out_shape=jax.ShapeDtypeStruct((G, C), jnp.float32),
        scratch_shapes=[pltpu.VMEM((G, H), jnp.float32)],
    )(p, p, r, b, batch3, wl, bl)


def kernel(x, edge_index, batch,
           W_rel0, b_rel0, W_root0,
           W_rel1, b_rel1, W_root1,
           W_rel2, b_rel2, W_root2,
           W_lin2, b_lin2):
    f32 = jnp.float32
    x = x.astype(f32)
    src = edge_index[0].astype(jnp.int32)
    dst = edge_index[1].astype(jnp.int32)
    pad = EP - E
    src2 = jnp.concatenate([src, jnp.zeros((pad,), jnp.int32)]).reshape(
        EP // CH, CH)
    dst2 = jnp.concatenate([dst, jnp.full((pad,), N, jnp.int32)]).reshape(
        EP // (CH // 2), CH // 2)
    batch3 = batch.astype(jnp.int32).reshape(GRID, 1, R)

    perm = jnp.asarray(_PERM)
    wrel0 = W_rel0.astype(f32)
    wroot0 = W_root0.astype(f32)[:, perm]
    b0 = b_rel0.astype(f32)[perm].reshape(1, H)
    wrel1 = W_rel1.astype(f32)[perm, :]
    wroot1 = W_root1.astype(f32)[perm][:, perm]
    b1 = b_rel1.astype(f32)[perm].reshape(1, H)
    wrel2 = W_rel2.astype(f32)[perm, :]
    wroot2 = W_root2.astype(f32)[perm][:, perm]
    b2 = b_rel2.astype(f32)[perm].reshape(1, H)
    wlin2 = W_lin2.astype(f32)[perm, :]
    bl = b_lin2.astype(f32).reshape(1, C)

    y16, r = _mm2(x, wrel0, wroot0)
    p = _sc_agg(_pack(y16), src2, dst2)
    y16, r = _combine(p, r, b0, wrel1, wroot1)
    p = _sc_agg(_pack(y16), src2, dst2)
    y16, r = _combine(p, r, b1, wrel2, wroot2)
    p = _sc_agg(_pack(y16), src2, dst2)
    return _final(p, r, b2, batch3, wlin2, bl)
